# Initial kernel scaffold; baseline (speedup 1.0000x reference)
import jax
import jax.numpy as jnp
from jax.experimental import pallas as pl


def kernel(x, base_weight, over_weight, override_mask):
    raise NotImplementedError("write your pallas kernel here")


# speedup vs baseline: 15.6445x; 15.6445x over previous
import functools

import jax
import jax.numpy as jnp
from jax import lax
from jax.experimental import pallas as pl
from jax.experimental.pallas import tpu as pltpu
from jax.experimental.pallas import tpu_sc as plsc

PAD = 0


_ROW_BLK = 2000


def _blend_body(b_ref, o_ref, m_ref, out_ref):
    b = b_ref[...]
    o = o_ref[...]
    m = m_ref[...]
    out_ref[...] = b * (1.0 - m) + o * m


def _blend_table(base_weight, over_weight, override_mask):
    V, D = base_weight.shape
    grid = V // _ROW_BLK
    return pl.pallas_call(
        _blend_body,
        grid=(grid,),
        in_specs=[
            pl.BlockSpec((_ROW_BLK, D), lambda i: (i, 0)),
            pl.BlockSpec((_ROW_BLK, D), lambda i: (i, 0)),
            pl.BlockSpec((_ROW_BLK, 1), lambda i: (i, 0)),
        ],
        out_specs=pl.BlockSpec((_ROW_BLK, D), lambda i: (i, 0)),
        out_shape=jax.ShapeDtypeStruct((V, D), jnp.float32),
    )(base_weight, over_weight, override_mask.reshape(V, 1))


_NC, _NS, _LANES = 2, 16, 16
_NW = _NC * _NS
_CHUNK = 1024
_SUB = 128


def _make_gather(N, V, D):
    per_w = N // _NW
    n_blk = per_w // _CHUNK
    mesh = plsc.VectorSubcoreMesh(core_axis_name="c", subcore_axis_name="s")

    @functools.partial(
        pl.kernel,
        mesh=mesh,
        compiler_params=pltpu.CompilerParams(use_tc_tiling_on_sc=False),
        out_type=[
            jax.ShapeDtypeStruct((N, D), jnp.float32),
            jax.ShapeDtypeStruct((N,), jnp.int32),
        ],
        scratch_types=[
            pltpu.VMEM((_CHUNK,), jnp.int32),
            pltpu.VMEM((_CHUNK, D), jnp.float32),
            pltpu.VMEM((_CHUNK,), jnp.int32),
            pltpu.SemaphoreType.DMA,
        ],
    )
    def gather_kernel(x_hbm, tbl_hbm, emb_hbm, msk_hbm, idx_v, rows_v, msk_v, dsem):
        wid = lax.axis_index("s") * _NC + lax.axis_index("c")
        base = wid * per_w

        def blk(i, carry):
            off = base + i * _CHUNK
            pltpu.sync_copy(x_hbm.at[pl.ds(off, _CHUNK)], idx_v)
            copies = [
                pltpu.async_copy(
                    tbl_hbm.at[idx_v.at[pl.ds(j * _SUB, _SUB)]],
                    rows_v.at[pl.ds(j * _SUB, _SUB)],
                    dsem,
                )
                for j in range(_CHUNK // _SUB)
            ]
            for j in range(_CHUNK // _LANES):
                iv = idx_v[pl.ds(j * _LANES, _LANES)]
                msk_v[pl.ds(j * _LANES, _LANES)] = jnp.minimum(iv, 1)
            pltpu.sync_copy(msk_v, msk_hbm.at[pl.ds(off, _CHUNK)])
            for c in copies:
                c.wait()
            pltpu.sync_copy(rows_v, emb_hbm.at[pl.ds(off, _CHUNK)])
            return carry

        lax.fori_loop(0, n_blk, blk, 0)

    return gather_kernel


def kernel(x, base_weight, over_weight, override_mask):
    B, L = x.shape
    V, D = base_weight.shape
    N = B * L
    xf = x.reshape(-1).astype(jnp.int32)
    tbl = _blend_table(base_weight, over_weight, override_mask)
    emb_flat, msk_flat = _make_gather(N, V, D)(xf, tbl)
    return emb_flat.reshape(B, L, D), msk_flat.reshape(B, L)


# speedup vs baseline: 21.5232x; 1.3758x over previous
import functools

import jax
import jax.numpy as jnp
from jax import lax
from jax.experimental import pallas as pl
from jax.experimental.pallas import tpu as pltpu
from jax.experimental.pallas import tpu_sc as plsc

PAD = 0


_ROW_BLK = 2000


def _blend_body(b_ref, o_ref, m_ref, out_ref):
    b = b_ref[...]
    o = o_ref[...]
    m = m_ref[...]
    out_ref[...] = b * (1.0 - m) + o * m


def _blend_table(base_weight, over_weight, override_mask):
    V, D = base_weight.shape
    grid = V // _ROW_BLK
    return pl.pallas_call(
        _blend_body,
        grid=(grid,),
        in_specs=[
            pl.BlockSpec((_ROW_BLK, D), lambda i: (i, 0)),
            pl.BlockSpec((_ROW_BLK, D), lambda i: (i, 0)),
            pl.BlockSpec((_ROW_BLK, 1), lambda i: (i, 0)),
        ],
        out_specs=pl.BlockSpec((_ROW_BLK, D), lambda i: (i, 0)),
        out_shape=jax.ShapeDtypeStruct((V, D), jnp.float32),
    )(base_weight, over_weight, override_mask.reshape(V, 1))


_SLAB = 128


def _make_transpose(B, L, D):
    LD = L * D

    def tr_body(in_hbm, out_ref, buf, sem):
        j = pl.program_id(0)
        for r in range(_SLAB):
            pltpu.make_async_copy(
                in_hbm.at[pl.ds((j * _SLAB + r) * LD, LD)], buf.at[r], sem
            ).start()
        for r in range(_SLAB):
            pltpu.make_async_copy(
                in_hbm.at[pl.ds((j * _SLAB + r) * LD, LD)], buf.at[r], sem
            ).wait()
        out_ref[...] = jnp.transpose(buf[...], (1, 0))

    return pl.pallas_call(
        tr_body,
        grid=(B // _SLAB,),
        in_specs=[pl.BlockSpec(memory_space=pl.ANY)],
        out_specs=pl.BlockSpec((LD, _SLAB), lambda j: (0, j)),
        out_shape=jax.ShapeDtypeStruct((LD, B), jnp.float32),
        scratch_shapes=[
            pltpu.VMEM((_SLAB, LD), jnp.float32),
            pltpu.SemaphoreType.DMA,
        ],
    )


def _transpose_out(emb_lin, B, L, D):
    embT2d = _make_transpose(B, L, D)(emb_lin.reshape(-1))
    return jnp.transpose(embT2d.reshape(L, D, B), (2, 0, 1))


_NC, _NS, _LANES = 2, 16, 16
_NW = _NC * _NS
_CHUNK = 1024
_SUB = 128


def _make_gather(N, V, D):
    per_w = N // _NW
    n_blk = per_w // _CHUNK
    mesh = plsc.VectorSubcoreMesh(core_axis_name="c", subcore_axis_name="s")

    @functools.partial(
        pl.kernel,
        mesh=mesh,
        compiler_params=pltpu.CompilerParams(use_tc_tiling_on_sc=False),
        out_type=[
            jax.ShapeDtypeStruct((N, D), jnp.float32),
            jax.ShapeDtypeStruct((N,), jnp.int32),
        ],
        scratch_types=[
            pltpu.VMEM((_CHUNK,), jnp.int32),
            pltpu.VMEM((_CHUNK, D), jnp.float32),
            pltpu.VMEM((_CHUNK,), jnp.int32),
            pltpu.SemaphoreType.DMA,
        ],
    )
    def gather_kernel(x_hbm, tbl_hbm, emb_hbm, msk_hbm, idx_v, rows_v, msk_v, dsem):
        wid = lax.axis_index("s") * _NC + lax.axis_index("c")
        base = wid * per_w

        def blk(i, carry):
            off = base + i * _CHUNK
            pltpu.sync_copy(x_hbm.at[pl.ds(off, _CHUNK)], idx_v)
            copies = [
                pltpu.async_copy(
                    tbl_hbm.at[idx_v.at[pl.ds(j * _SUB, _SUB)]],
                    rows_v.at[pl.ds(j * _SUB, _SUB)],
                    dsem,
                )
                for j in range(_CHUNK // _SUB)
            ]
            for j in range(_CHUNK // _LANES):
                iv = idx_v[pl.ds(j * _LANES, _LANES)]
                msk_v[pl.ds(j * _LANES, _LANES)] = jnp.minimum(iv, 1)
            pltpu.sync_copy(msk_v, msk_hbm.at[pl.ds(off, _CHUNK)])
            for c in copies:
                c.wait()
            pltpu.sync_copy(rows_v, emb_hbm.at[pl.ds(off, _CHUNK)])
            return carry

        lax.fori_loop(0, n_blk, blk, 0)

    return gather_kernel


def kernel(x, base_weight, over_weight, override_mask):
    B, L = x.shape
    V, D = base_weight.shape
    N = B * L
    xf = x.reshape(-1).astype(jnp.int32)
    tbl = _blend_table(base_weight, over_weight, override_mask)
    emb_flat, msk_flat = _make_gather(N, V, D)(xf, tbl)
    emb = _transpose_out(emb_flat, B, L, D)
    return emb, msk_flat.reshape(B, L)
